# baseline (speedup 1.0000x reference)
import functools

import jax
import jax.numpy as jnp
from jax import lax
from jax.experimental import pallas as pl
from jax.experimental.pallas import tpu as pltpu
from jax.experimental.pallas import tpu_sc as plsc

N = 10000
E = 320000
D = 128
H = 64
ED = 16
PW = 128

NC, NS = 2, 16
NW = NC * NS
NPAD = 10240
EPAD = 327680
NHALF = 2
EH = EPAD // NHALF
EWH = EH // NW
GRP = 128
NGH = EWH // GRP
NGROW = EPAD // GRP
SSUP = 128
SNH = EWH // SSUP

_mesh = plsc.VectorSubcoreMesh(core_axis_name="c", subcore_axis_name="s")


def _pre_body(x_ref, pos_ref, w1a_ref, w1b_ref, g1_ref, g2_ref):
    xx = x_ref[...]
    p = pos_ref[...]
    z = jnp.zeros((xx.shape[0], PW - H - 3), jnp.float32)
    p1 = jnp.dot(xx, w1a_ref[...], preferred_element_type=jnp.float32)
    p2 = jnp.dot(xx, w1b_ref[...], preferred_element_type=jnp.float32)
    g1_ref[...] = jnp.concatenate([p1, p, z], axis=1)
    g2_ref[...] = jnp.concatenate([p2, -p, z], axis=1)


def _precompute(x_p, pos_p, w1a, w1b):
    bn = 5120
    return pl.pallas_call(
        _pre_body,
        grid=(NPAD // bn,),
        in_specs=[
            pl.BlockSpec((bn, D), lambda i: (i, 0)),
            pl.BlockSpec((bn, 3), lambda i: (i, 0)),
            pl.BlockSpec((D, H), lambda i: (0, 0)),
            pl.BlockSpec((D, H), lambda i: (0, 0)),
        ],
        out_specs=[
            pl.BlockSpec((bn, PW), lambda i: (i, 0)),
            pl.BlockSpec((bn, PW), lambda i: (i, 0)),
        ],
        out_shape=[
            jax.ShapeDtypeStruct((NPAD, PW), jnp.float32),
            jax.ShapeDtypeStruct((NPAD, PW), jnp.float32),
        ],
    )(x_p, pos_p, w1a, w1b)


def _make_gather(half):
    @functools.partial(
        pl.kernel,
        mesh=_mesh,
        out_type=jax.ShapeDtypeStruct((EH, PW), jnp.float32),
        scratch_types=[
            pltpu.VMEM((NGH, GRP), jnp.int32),
            pltpu.VMEM((NGH, GRP), jnp.int32),
            pltpu.VMEM((3, GRP, PW), jnp.float32),
            pltpu.VMEM((3, GRP, PW), jnp.float32),
            pltpu.SemaphoreType.DMA,
            pltpu.SemaphoreType.DMA,
            pltpu.SemaphoreType.DMA,
            pltpu.SemaphoreType.DMA,
            pltpu.SemaphoreType.DMA,
            pltpu.SemaphoreType.DMA,
        ],
    )
    def _gather(g1_hbm, g2_hbm, idx_hbm, out_hbm,
                ridx_all, cidx_all, g1b, tdb, sg0, sg1, sg2, so0, so1, so2):
        wid = lax.axis_index("s") * NC + lax.axis_index("c")
        gbase = half * (NGROW // NHALF) + wid * NGH
        pltpu.sync_copy(idx_hbm.at[pl.ds(gbase, NGH)], ridx_all)
        pltpu.sync_copy(idx_hbm.at[pl.ds(NGROW + gbase, NGH)], cidx_all)

        g_sems = (sg0, sg1, sg2)
        o_sems = (so0, so1, so2)

        def fire_gather(s, slot):
            return (pltpu.async_copy(g1_hbm.at[ridx_all.at[s]], g1b.at[slot],
                                     g_sems[slot]),
                    pltpu.async_copy(g2_hbm.at[cidx_all.at[s]], tdb.at[slot],
                                     g_sems[slot]))

        def out_region(s):
            return out_hbm.at[pl.ds(wid * EWH + s * GRP, GRP)]

        def add_in(slot):
            def rows(u, carry):
                for k in range(2):
                    r = u * 2 + k
                    for v in range(5):
                        sl = pl.ds(v * 16, 16)
                        tdb[slot, r, sl] = g1b[slot, r, sl] + tdb[slot, r, sl]
                return carry
            lax.fori_loop(0, GRP // 2, rows, 0)

        pend_g = {0: fire_gather(0, 0), 1: fire_gather(1, 1)}
        pend_o = {}
        for s in range(NGH):
            slot = s % 3
            if s + 2 < NGH:
                nslot = (s + 2) % 3
                if nslot in pend_o:
                    pend_o.pop(nslot).wait()
                pend_g[nslot] = fire_gather(s + 2, nslot)
            for h in pend_g.pop(slot):
                h.wait()
            add_in(slot)
            pend_o[slot] = pltpu.async_copy(tdb.at[slot], out_region(s),
                                            o_sems[slot])
        for h in pend_o.values():
            h.wait()
    return _gather


def _edge_body(td_ref, ea_ref, w1c_ref, w1d_ref, b1_ref, w2_ref, b2_ref,
               wc1_ref, bc1_ref, wc2_ref, bc2_ref, out_ref):
    td = td_ref[...]
    t = td[:, :H]
    diff = td[:, H:H + 3]
    dist = jnp.sqrt(jnp.sum(diff * diff, axis=1, keepdims=True))
    u = (t + dist * w1c_ref[...] + b1_ref[...]
         + jnp.dot(ea_ref[...], w1d_ref[...], preferred_element_type=jnp.float32))
    h1 = u * jax.nn.sigmoid(u)
    m = jnp.dot(h1, w2_ref[...], preferred_element_type=jnp.float32) + b2_ref[...]
    msg = m * jax.nn.sigmoid(m)
    cw = jnp.dot(msg, wc1_ref[...], preferred_element_type=jnp.float32) + bc1_ref[...]
    cw = cw * jax.nn.sigmoid(cw)
    cw = jnp.dot(cw, wc2_ref[...], preferred_element_type=jnp.float32) + bc2_ref[...]
    cd = diff * cw
    z = jnp.zeros((td.shape[0], PW - H - 3), jnp.float32)
    out_ref[...] = jnp.concatenate([msg, cd, z], axis=1)


def _edge_mlp(td, ea_p, w1c, w1d, b1, w2, b2, wc1, bc1, wc2, bc2):
    be = 8192
    full = lambda shape: pl.BlockSpec(shape, lambda i: (0, 0))
    return pl.pallas_call(
        _edge_body,
        grid=(EH // be,),
        in_specs=[
            pl.BlockSpec((be, PW), lambda i: (i, 0)),
            pl.BlockSpec((be, ED), lambda i: (i, 0)),
            full((1, H)), full((ED, H)), full((1, H)),
            full((H, H)), full((1, H)),
            full((H, H)), full((1, H)),
            full((H, 1)), full((1, 1)),
        ],
        out_specs=pl.BlockSpec((be, PW), lambda i: (i, 0)),
        out_shape=jax.ShapeDtypeStruct((EH, PW), jnp.float32),
    )(td, ea_p, w1c, w1d, b1, w2, b2, wc1, bc1, wc2, bc2)


@functools.partial(
    pl.kernel,
    mesh=_mesh,
    out_type=jax.ShapeDtypeStruct((NC, NPAD, PW), jnp.float32),
    scratch_types=[
        pltpu.VMEM((NHALF * NGH, GRP), jnp.int32),
        pltpu.VMEM((2, SSUP, PW), jnp.float32),
        pltpu.VMEM_SHARED((NPAD, PW), jnp.float32),
        pltpu.SemaphoreType.DMA,
        pltpu.SemaphoreType.DMA,
    ],
)
def _sc_scatter(md0_hbm, md1_hbm, idx_hbm, zeros_hbm, out_hbm,
                didx_all, mbuf, acc, sl0, sl1):
    cid = lax.axis_index("c")
    sid = lax.axis_index("s")
    wid = sid * NC + cid
    stripe = NPAD // NS
    pltpu.sync_copy(zeros_hbm.at[pl.ds(sid * stripe, stripe)],
                    acc.at[pl.ds(sid * stripe, stripe)])
    for h in range(NHALF):
        pltpu.sync_copy(
            idx_hbm.at[pl.ds(h * (NGROW // NHALF) + wid * NGH, NGH)],
            didx_all.at[pl.ds(h * NGH, NGH)])
    plsc.subcore_barrier()

    mds = (md0_hbm, md1_hbm)

    def md_region(s):
        return mds[s // SNH].at[pl.ds(wid * EWH + (s % SNH) * SSUP, SSUP)]

    l_sems = (sl0, sl1)
    pend_l = {0: pltpu.async_copy(md_region(0), mbuf.at[0], sl0)}
    for s in range(NHALF * SNH):
        slot = s % 2
        if s + 1 < NHALF * SNH:
            pend_l[1 - slot] = pltpu.async_copy(md_region(s + 1),
                                                mbuf.at[1 - slot],
                                                l_sems[1 - slot])
        pend_l.pop(slot).wait()
        pltpu.sync_copy(mbuf.at[slot], acc.at[didx_all.at[s]], add=True)
    plsc.subcore_barrier()
    pltpu.sync_copy(acc.at[pl.ds(sid * stripe, stripe)],
                    out_hbm.at[cid, pl.ds(sid * stripe, stripe)])


def _node_body(acc_ref, x_ref, pos_ref, wn1a_ref, wn1b_ref, bn1_ref,
               wn2_ref, bn2_ref, xout_ref, pout_ref):
    a = acc_ref[0] + acc_ref[1]
    agg = a[:, :H]
    cagg = a[:, H:H + 3]
    xx = x_ref[...]
    u = (jnp.dot(xx, wn1a_ref[...], preferred_element_type=jnp.float32)
         + jnp.dot(agg, wn1b_ref[...], preferred_element_type=jnp.float32)
         + bn1_ref[...])
    h = u * jax.nn.sigmoid(u)
    xout_ref[...] = jnp.dot(h, wn2_ref[...], preferred_element_type=jnp.float32) + bn2_ref[...]
    pout_ref[...] = pos_ref[...] + cagg


def _node_mlp(accs, x_p, pos_p, wn1a, wn1b, bn1, wn2, bn2):
    bn = 1280
    full = lambda shape: pl.BlockSpec(shape, lambda i: tuple(0 for _ in shape))
    return pl.pallas_call(
        _node_body,
        grid=(NPAD // bn,),
        in_specs=[
            pl.BlockSpec((NC, bn, PW), lambda i: (0, i, 0)),
            pl.BlockSpec((bn, D), lambda i: (i, 0)),
            pl.BlockSpec((bn, 3), lambda i: (i, 0)),
            full((D, H)), full((H, H)), full((1, H)),
            full((H, D)), full((1, D)),
        ],
        out_specs=[
            pl.BlockSpec((bn, D), lambda i: (i, 0)),
            pl.BlockSpec((bn, 3), lambda i: (i, 0)),
        ],
        out_shape=[
            jax.ShapeDtypeStruct((NPAD, D), jnp.float32),
            jax.ShapeDtypeStruct((NPAD, 3), jnp.float32),
        ],
    )(accs, x_p, pos_p, wn1a, wn1b, bn1, wn2, bn2)


def kernel(x, pos, edge_index, edge_attr, W1, b1, W2, b2,
           Wn1, bn1, Wn2, bn2, Wc1, bc1, Wc2, bc2):
    row = edge_index[0].astype(jnp.int32)
    col = edge_index[1].astype(jnp.int32)

    x_p = jnp.pad(x, ((0, NPAD - N), (0, 0)))
    pos_p = jnp.pad(pos, ((0, NPAD - N), (0, 0)))
    pad_idx = N + (jnp.arange(EPAD - E, dtype=jnp.int32) % (NPAD - N))
    idx_all = jnp.concatenate([
        jnp.concatenate([row, pad_idx]).reshape(NGROW, GRP),
        jnp.concatenate([col, pad_idx]).reshape(NGROW, GRP),
    ])
    ea_p = jnp.pad(edge_attr, ((0, EPAD - E), (0, 0)))

    w1a = W1[:D]
    w1b = W1[D:2 * D]
    w1c = W1[2 * D:2 * D + 1]
    w1d = W1[2 * D + 1:]

    g1, g2 = _precompute(x_p, pos_p, w1a, w1b)
    zeros = jnp.zeros((NPAD, PW), jnp.float32)
    b1r, b2r = b1.reshape(1, H), b2.reshape(1, H)
    bc1r, bc2r = bc1.reshape(1, H), bc2.reshape(1, 1)

    mds = []
    for h in range(NHALF):
        td = _make_gather(h)(g1, g2, idx_all)
        mds.append(_edge_mlp(td, ea_p[h * EH:(h + 1) * EH], w1c, w1d, b1r,
                             W2, b2r, Wc1, bc1r, Wc2, bc2r))
    acc = _sc_scatter(mds[0], mds[1], idx_all, zeros)

    x_new_p, pos_new_p = _node_mlp(acc, x_p, pos_p, Wn1[:D], Wn1[D:],
                                   bn1.reshape(1, H), Wn2, bn2.reshape(1, D))
    return (x_new_p[:N], pos_new_p[:N])


# speedup vs baseline: 1.0330x; 1.0330x over previous
import functools

import jax
import jax.numpy as jnp
from jax import lax
from jax.experimental import pallas as pl
from jax.experimental.pallas import tpu as pltpu
from jax.experimental.pallas import tpu_sc as plsc

N = 10000
E = 320000
D = 128
H = 64
ED = 16
PW = 128

NC, NS = 2, 16
NW = NC * NS
NPAD = 10240
EPAD = 327680
NHALF = 2
EH = EPAD // NHALF
EWH = EH // NW
GRP = 128
NGH = EWH // GRP
NGROW = EPAD // GRP
SSUP = 128
SNH = EWH // SSUP

_mesh = plsc.VectorSubcoreMesh(core_axis_name="c", subcore_axis_name="s")


def _pre_body(x_ref, pos_ref, w1a_ref, w1b_ref, g1_ref, g2_ref):
    xx = x_ref[...]
    p = pos_ref[...]
    z = jnp.zeros((xx.shape[0], PW - H - 3), jnp.float32)
    p1 = jnp.dot(xx, w1a_ref[...], preferred_element_type=jnp.float32)
    p2 = jnp.dot(xx, w1b_ref[...], preferred_element_type=jnp.float32)
    g1_ref[...] = jnp.concatenate([p1, p, z], axis=1)
    g2_ref[...] = jnp.concatenate([p2, -p, z], axis=1)


def _precompute(x_p, pos_p, w1a, w1b):
    bn = 5120
    return pl.pallas_call(
        _pre_body,
        grid=(NPAD // bn,),
        in_specs=[
            pl.BlockSpec((bn, D), lambda i: (i, 0)),
            pl.BlockSpec((bn, 3), lambda i: (i, 0)),
            pl.BlockSpec((D, H), lambda i: (0, 0)),
            pl.BlockSpec((D, H), lambda i: (0, 0)),
        ],
        out_specs=[
            pl.BlockSpec((bn, PW), lambda i: (i, 0)),
            pl.BlockSpec((bn, PW), lambda i: (i, 0)),
        ],
        out_shape=[
            jax.ShapeDtypeStruct((NPAD, PW), jnp.float32),
            jax.ShapeDtypeStruct((NPAD, PW), jnp.float32),
        ],
    )(x_p, pos_p, w1a, w1b)


def _make_gather(half):
    @functools.partial(
        pl.kernel,
        mesh=_mesh,
        out_type=jax.ShapeDtypeStruct((EH, PW), jnp.float32),
        scratch_types=[
            pltpu.VMEM((NGH, GRP), jnp.int32),
            pltpu.VMEM((NGH, GRP), jnp.int32),
            pltpu.VMEM((3, GRP, PW), jnp.float32),
            pltpu.VMEM((3, GRP, PW), jnp.float32),
            pltpu.SemaphoreType.DMA,
            pltpu.SemaphoreType.DMA,
            pltpu.SemaphoreType.DMA,
            pltpu.SemaphoreType.DMA,
            pltpu.SemaphoreType.DMA,
            pltpu.SemaphoreType.DMA,
        ],
    )
    def _gather(g1_hbm, g2_hbm, idx_hbm, out_hbm,
                ridx_all, cidx_all, g1b, tdb, sg0, sg1, sg2, so0, so1, so2):
        wid = lax.axis_index("s") * NC + lax.axis_index("c")
        gbase = half * (NGROW // NHALF) + wid * NGH
        pltpu.sync_copy(idx_hbm.at[pl.ds(gbase, NGH)], ridx_all)
        pltpu.sync_copy(idx_hbm.at[pl.ds(NGROW + gbase, NGH)], cidx_all)

        g_sems = (sg0, sg1, sg2)
        o_sems = (so0, so1, so2)

        def fire_gather(s, slot):
            return (pltpu.async_copy(g1_hbm.at[ridx_all.at[s]], g1b.at[slot],
                                     g_sems[slot]),
                    pltpu.async_copy(g2_hbm.at[cidx_all.at[s]], tdb.at[slot],
                                     g_sems[slot]))

        def out_region(s):
            return out_hbm.at[pl.ds(wid * EWH + s * GRP, GRP)]

        def add_in(slot):
            def rows(u, carry):
                for k in range(2):
                    r = u * 2 + k
                    for v in range(5):
                        sl = pl.ds(v * 16, 16)
                        tdb[slot, r, sl] = g1b[slot, r, sl] + tdb[slot, r, sl]
                return carry
            lax.fori_loop(0, GRP // 2, rows, 0)

        pend_g = {0: fire_gather(0, 0), 1: fire_gather(1, 1)}
        pend_o = {}
        for s in range(NGH):
            slot = s % 3
            if s + 2 < NGH:
                nslot = (s + 2) % 3
                if nslot in pend_o:
                    pend_o.pop(nslot).wait()
                pend_g[nslot] = fire_gather(s + 2, nslot)
            for h in pend_g.pop(slot):
                h.wait()
            add_in(slot)
            pend_o[slot] = pltpu.async_copy(tdb.at[slot], out_region(s),
                                            o_sems[slot])
        for h in pend_o.values():
            h.wait()
    return _gather


def _edge_body(td_ref, ea_ref, w1c_ref, w1d_ref, b1_ref, w2_ref, b2_ref,
               wc1_ref, bc1_ref, wc2_ref, bc2_ref, out_ref):
    td = td_ref[...]
    t = td[:, :H]
    diff = td[:, H:H + 3]
    dist = jnp.sqrt(jnp.sum(diff * diff, axis=1, keepdims=True))
    u = (t + dist * w1c_ref[...] + b1_ref[...]
         + jnp.dot(ea_ref[...], w1d_ref[...], preferred_element_type=jnp.float32))
    h1 = u * jax.nn.sigmoid(u)
    m = jnp.dot(h1, w2_ref[...], preferred_element_type=jnp.float32) + b2_ref[...]
    msg = m * jax.nn.sigmoid(m)
    cw = jnp.dot(msg, wc1_ref[...], preferred_element_type=jnp.float32) + bc1_ref[...]
    cw = cw * jax.nn.sigmoid(cw)
    cw = jnp.dot(cw, wc2_ref[...], preferred_element_type=jnp.float32) + bc2_ref[...]
    cd = diff * cw
    z = jnp.zeros((td.shape[0], PW - H - 3), jnp.float32)
    out_ref[...] = jnp.concatenate([msg, cd, z], axis=1)


def _edge_mlp(td, ea_p, w1c, w1d, b1, w2, b2, wc1, bc1, wc2, bc2):
    be = 8192
    full = lambda shape: pl.BlockSpec(shape, lambda i: (0, 0))
    return pl.pallas_call(
        _edge_body,
        grid=(EH // be,),
        in_specs=[
            pl.BlockSpec((be, PW), lambda i: (i, 0)),
            pl.BlockSpec((be, ED), lambda i: (i, 0)),
            full((1, H)), full((ED, H)), full((1, H)),
            full((H, H)), full((1, H)),
            full((H, H)), full((1, H)),
            full((H, 1)), full((1, 1)),
        ],
        out_specs=pl.BlockSpec((be, PW), lambda i: (i, 0)),
        out_shape=jax.ShapeDtypeStruct((EH, PW), jnp.float32),
    )(td, ea_p, w1c, w1d, b1, w2, b2, wc1, bc1, wc2, bc2)


def _make_scatter(half):
    @functools.partial(
        pl.kernel,
        mesh=_mesh,
        out_type=jax.ShapeDtypeStruct((NC, NPAD, PW), jnp.float32),
        scratch_types=[
            pltpu.VMEM((NGH, GRP), jnp.int32),
            pltpu.VMEM((2, SSUP, PW), jnp.float32),
            pltpu.VMEM_SHARED((NPAD, PW), jnp.float32),
            pltpu.SemaphoreType.DMA,
            pltpu.SemaphoreType.DMA,
        ],
    )
    def _scatter(md_hbm, idx_hbm, zeros_hbm, out_hbm, didx_all, mbuf, acc,
                 sl0, sl1):
        cid = lax.axis_index("c")
        sid = lax.axis_index("s")
        wid = sid * NC + cid
        stripe = NPAD // NS
        pltpu.sync_copy(zeros_hbm.at[pl.ds(sid * stripe, stripe)],
                        acc.at[pl.ds(sid * stripe, stripe)])
        gbase = half * (NGROW // NHALF) + wid * NGH
        pltpu.sync_copy(idx_hbm.at[pl.ds(gbase, NGH)], didx_all)
        plsc.subcore_barrier()

        def md_region(s):
            return md_hbm.at[pl.ds(wid * EWH + s * SSUP, SSUP)]

        l_sems = (sl0, sl1)
        pend_l = {0: pltpu.async_copy(md_region(0), mbuf.at[0], sl0)}
        for s in range(SNH):
            slot = s % 2
            if s + 1 < SNH:
                pend_l[1 - slot] = pltpu.async_copy(md_region(s + 1),
                                                    mbuf.at[1 - slot],
                                                    l_sems[1 - slot])
            pend_l.pop(slot).wait()
            pltpu.sync_copy(mbuf.at[slot], acc.at[didx_all.at[s]], add=True)
        plsc.subcore_barrier()
        pltpu.sync_copy(acc.at[pl.ds(sid * stripe, stripe)],
                        out_hbm.at[cid, pl.ds(sid * stripe, stripe)])
    return _scatter


def _node_body(acc_ref, x_ref, pos_ref, wn1a_ref, wn1b_ref, bn1_ref,
               wn2_ref, bn2_ref, xout_ref, pout_ref):
    a = (acc_ref[0] + acc_ref[1]) + (acc_ref[2] + acc_ref[3])
    agg = a[:, :H]
    cagg = a[:, H:H + 3]
    xx = x_ref[...]
    u = (jnp.dot(xx, wn1a_ref[...], preferred_element_type=jnp.float32)
         + jnp.dot(agg, wn1b_ref[...], preferred_element_type=jnp.float32)
         + bn1_ref[...])
    h = u * jax.nn.sigmoid(u)
    xout_ref[...] = jnp.dot(h, wn2_ref[...], preferred_element_type=jnp.float32) + bn2_ref[...]
    pout_ref[...] = pos_ref[...] + cagg


def _node_mlp(accs, x_p, pos_p, wn1a, wn1b, bn1, wn2, bn2):
    bn = 1280
    full = lambda shape: pl.BlockSpec(shape, lambda i: tuple(0 for _ in shape))
    return pl.pallas_call(
        _node_body,
        grid=(NPAD // bn,),
        in_specs=[
            pl.BlockSpec((2 * NC, bn, PW), lambda i: (0, i, 0)),
            pl.BlockSpec((bn, D), lambda i: (i, 0)),
            pl.BlockSpec((bn, 3), lambda i: (i, 0)),
            full((D, H)), full((H, H)), full((1, H)),
            full((H, D)), full((1, D)),
        ],
        out_specs=[
            pl.BlockSpec((bn, D), lambda i: (i, 0)),
            pl.BlockSpec((bn, 3), lambda i: (i, 0)),
        ],
        out_shape=[
            jax.ShapeDtypeStruct((NPAD, D), jnp.float32),
            jax.ShapeDtypeStruct((NPAD, 3), jnp.float32),
        ],
    )(accs, x_p, pos_p, wn1a, wn1b, bn1, wn2, bn2)


def kernel(x, pos, edge_index, edge_attr, W1, b1, W2, b2,
           Wn1, bn1, Wn2, bn2, Wc1, bc1, Wc2, bc2):
    row = edge_index[0].astype(jnp.int32)
    col = edge_index[1].astype(jnp.int32)

    x_p = jnp.pad(x, ((0, NPAD - N), (0, 0)))
    pos_p = jnp.pad(pos, ((0, NPAD - N), (0, 0)))
    pad_idx = N + (jnp.arange(EPAD - E, dtype=jnp.int32) % (NPAD - N))
    idx_all = jnp.concatenate([
        jnp.concatenate([row, pad_idx]).reshape(NGROW, GRP),
        jnp.concatenate([col, pad_idx]).reshape(NGROW, GRP),
    ])
    ea_p = jnp.pad(edge_attr, ((0, EPAD - E), (0, 0)))

    w1a = W1[:D]
    w1b = W1[D:2 * D]
    w1c = W1[2 * D:2 * D + 1]
    w1d = W1[2 * D + 1:]

    g1, g2 = _precompute(x_p, pos_p, w1a, w1b)
    zeros = jnp.zeros((NPAD, PW), jnp.float32)
    b1r, b2r = b1.reshape(1, H), b2.reshape(1, H)
    bc1r, bc2r = bc1.reshape(1, H), bc2.reshape(1, 1)

    accs = []
    for h in range(NHALF):
        td = _make_gather(h)(g1, g2, idx_all)
        md = _edge_mlp(td, ea_p[h * EH:(h + 1) * EH], w1c, w1d, b1r,
                       W2, b2r, Wc1, bc1r, Wc2, bc2r)
        accs.append(_make_scatter(h)(md, idx_all, zeros))
    acc = jnp.concatenate(accs)

    x_new_p, pos_new_p = _node_mlp(acc, x_p, pos_p, Wn1[:D], Wn1[D:],
                                   bn1.reshape(1, H), Wn2, bn2.reshape(1, D))
    return (x_new_p[:N], pos_new_p[:N])
